# baseline (speedup 1.0000x reference)
import functools

import jax
import jax.numpy as jnp
from jax import lax
from jax.experimental import pallas as pl
from jax.experimental.pallas import tpu as pltpu
from jax.experimental.pallas import tpu_sc as plsc

VOCAB = 1000000
D_MODEL = 64
BATCH = 4096
SEQ = 200

_INFO = plsc.get_sparse_core_info()
_NC = _INFO.num_cores
_NS = _INFO.num_subcores
_NW = _NC * _NS
_L = 16

_N = BATCH * SEQ
_C = 128
_PER_W = _N // _NW // _C
_NB = 3


def _make_kernel():
    mesh = plsc.VectorSubcoreMesh(core_axis_name="c", subcore_axis_name="s")

    @functools.partial(
        pl.kernel,
        mesh=mesh,
        out_type=jax.ShapeDtypeStruct((SEQ, D_MODEL, BATCH), jnp.float32),
        scratch_types=(
            [pltpu.VMEM((_PER_W, _C), jnp.int32)]
            + [pltpu.VMEM((_C,), jnp.int32) for _ in range(_NB)]
            + [pltpu.VMEM((_C, 2 * D_MODEL), jnp.float32) for _ in range(_NB)]
            + [pltpu.VMEM((D_MODEL, _C), jnp.float32) for _ in range(_NB)]
            + [pltpu.SemaphoreType.DMA for _ in range(2 * _NB)]
        ),
        compiler_params=pltpu.CompilerParams(use_tc_tiling_on_sc=True,
                                             needs_layout_passes=False),
    )
    def body(idx_hbm, table_hbm, out_hbm, idx_v, *rest):
        ih = rest[:_NB]
        gb = rest[_NB:2 * _NB]
        st = rest[2 * _NB:3 * _NB]
        sg = rest[3 * _NB:3 * _NB + _NB]
        sw = rest[3 * _NB + _NB:]
        wid = lax.axis_index("s") * _NC + lax.axis_index("c")

        pltpu.sync_copy(idx_hbm.at[wid], idx_v)

        def prep(r, b):
            for v in range(_C // _L):
                x = idx_v[r, pl.ds(v * _L, _L)]
                ih[b][pl.ds(v * _L, _L)] = jax.lax.shift_right_logical(x, 1)

        def g_start(b):
            pltpu.async_copy(table_hbm.at[ih[b]], gb[b], sg[b])

        def g_wait(b):
            pltpu.make_async_copy(table_hbm.at[ih[b]], gb[b], sg[b]).wait()

        def out_slc(r):
            base = wid * (_PER_W * _C) + r * _C
            s = base // BATCH
            b0 = lax.rem(base, BATCH)
            return out_hbm.at[s, :, pl.ds(b0, _C)]

        def w_start(r, b):
            pltpu.async_copy(st[b], out_slc(r), sw[b])

        def w_wait(r, b):
            pltpu.make_async_copy(st[b], out_slc(r), sw[b]).wait()

        def transpose_select(r, b):
            jvs = []
            pvs = []
            for v in range(_C // _L):
                jvs.append(lax.iota(jnp.int32, _L) + v * _L)
                pvs.append(jax.lax.shift_left(
                    jnp.bitwise_and(idx_v[r, pl.ds(v * _L, _L)], 1), 6))

            def dbody(d, carry):
                for v in range(_C // _L):
                    st[b][d, pl.ds(v * _L, _L)] = plsc.load_gather(
                        gb[b], [jvs[v], pvs[v] + d])
                return carry

            lax.fori_loop(0, D_MODEL, dbody, 0)

        for b in range(_NB):
            prep(b, b)
            g_start(b)

        def step(j, carry):
            base = j * _NB
            for b in range(_NB):
                r = base + b
                g_wait(b)
                transpose_select(r, b)
                w_start(r, b)
                pb = (b - 1) % _NB
                pr = r + _NB - 1

                @pl.when(jnp.logical_and(pr >= _NB, pr < _PER_W))
                def _():
                    w_wait(pr - _NB, pb)
                    prep(pr, pb)
                    g_start(pb)
            return carry

        lax.fori_loop(0, _PER_W // _NB, step, 0)

        for r in range(_PER_W - _PER_W % _NB, _PER_W):
            b = r % _NB
            g_wait(b)
            transpose_select(r, b)
            w_start(r, b)
        for r in range(_PER_W - _NB, _PER_W):
            w_wait(r, r % _NB)

    return body


_sc_kernel = _make_kernel()


def kernel(inp, table):
    idx3 = jnp.transpose(inp).reshape(_NW, _PER_W, _C)
    table2 = table.reshape(VOCAB // 2, 2 * D_MODEL)
    out3 = _sc_kernel(idx3, table2)
    return jnp.transpose(out3, (0, 2, 1))


# speedup vs baseline: 1.6471x; 1.6471x over previous
import functools

import jax
import jax.numpy as jnp
from jax import lax
from jax.experimental import pallas as pl
from jax.experimental.pallas import tpu as pltpu
from jax.experimental.pallas import tpu_sc as plsc

VOCAB = 1000000
D_MODEL = 64
BATCH = 4096
SEQ = 200

_INFO = plsc.get_sparse_core_info()
_NC = _INFO.num_cores
_NS = _INFO.num_subcores
_NW = _NC * _NS
_L = 16

_N = BATCH * SEQ
_C = 128
_PER_W = _N // _NW // _C
_NB = 3


def _make_kernel():
    mesh = plsc.VectorSubcoreMesh(core_axis_name="c", subcore_axis_name="s")

    @functools.partial(
        pl.kernel,
        mesh=mesh,
        out_type=jax.ShapeDtypeStruct((SEQ, BATCH, D_MODEL), jnp.float32),
        scratch_types=(
            [pltpu.VMEM((_PER_W, _C), jnp.int32)]
            + [pltpu.VMEM((_C,), jnp.int32) for _ in range(_NB)]
            + [pltpu.VMEM((_C, 2 * D_MODEL), jnp.float32) for _ in range(_NB)]
            + [pltpu.VMEM((_C, D_MODEL), jnp.float32) for _ in range(_NB)]
            + [pltpu.SemaphoreType.DMA for _ in range(2 * _NB)]
        ),
        compiler_params=pltpu.CompilerParams(use_tc_tiling_on_sc=True,
                                             needs_layout_passes=False),
    )
    def body(idx_hbm, table_hbm, out_hbm, idx_v, *rest):
        ih = rest[:_NB]
        gb = rest[_NB:2 * _NB]
        st = rest[2 * _NB:3 * _NB]
        sg = rest[3 * _NB:3 * _NB + _NB]
        sw = rest[3 * _NB + _NB:]
        wid = lax.axis_index("s") * _NC + lax.axis_index("c")

        pltpu.sync_copy(idx_hbm.at[wid], idx_v)

        def prep(r, b):
            for v in range(_C // _L):
                ih[b][pl.ds(v * _L, _L)] = idx_v[r, pl.ds(v * _L, _L)]

        def g_start(b):
            pltpu.async_copy(table_hbm.at[ih[b]], gb[b], sg[b])

        def g_wait(b):
            pltpu.make_async_copy(table_hbm.at[ih[b]], gb[b], sg[b]).wait()

        def out_slc(r):
            base = wid * (_PER_W * _C) + r * _C
            s = base // BATCH
            b0 = pl.multiple_of(lax.rem(base, BATCH), 8)
            return out_hbm.at[s, pl.ds(b0, _C), :]

        def w_start(r, b):
            pltpu.async_copy(st[b], out_slc(r), sw[b])

        def w_wait(r, b):
            pltpu.make_async_copy(st[b], out_slc(r), sw[b]).wait()

        def compact64(b):
            def jbody(j2, carry):
                j0 = j2 * 2
                vals = []
                for jj in range(2):
                    for v in range(D_MODEL // _L):
                        vals.append(gb[b][j0 + jj, pl.ds(v * _L, _L)])
                i = 0
                for jj in range(2):
                    for v in range(D_MODEL // _L):
                        st[b][j0 + jj, pl.ds(v * _L, _L)] = vals[i]
                        i += 1
                return carry

            lax.fori_loop(0, _C // 2, jbody, 0)

        for b in range(_NB):
            prep(b, b)
            g_start(b)

        def step(j, carry):
            base = j * _NB
            for b in range(_NB):
                r = base + b
                g_wait(b)
                compact64(b)
                w_start(r, b)
                pb = (b - 1) % _NB
                pr = r + _NB - 1

                @pl.when(jnp.logical_and(pr >= _NB, pr < _PER_W))
                def _():
                    w_wait(pr - _NB, pb)
                    prep(pr, pb)
                    g_start(pb)
            return carry

        lax.fori_loop(0, _PER_W // _NB, step, 0)

        for r in range(_PER_W - _PER_W % _NB, _PER_W):
            b = r % _NB
            g_wait(b)
            compact64(b)
            w_start(r, b)
        for r in range(_PER_W - _NB, _PER_W):
            w_wait(r, r % _NB)

    return body


_sc_kernel = _make_kernel()


def kernel(inp, table):
    idx3 = jnp.transpose(inp).reshape(_NW, _PER_W, _C)
    tableP = jnp.pad(table, ((0, 0), (0, D_MODEL)))
    return _sc_kernel(idx3, tableP)


# speedup vs baseline: 1.8662x; 1.1331x over previous
import functools

import jax
import jax.numpy as jnp
from jax import lax
from jax.experimental import pallas as pl
from jax.experimental.pallas import tpu as pltpu
from jax.experimental.pallas import tpu_sc as plsc

VOCAB = 1000000
D_MODEL = 64
BATCH = 4096
SEQ = 200

_INFO = plsc.get_sparse_core_info()
_NC = _INFO.num_cores
_NS = _INFO.num_subcores
_NW = _NC * _NS
_L = 16

_N = BATCH * SEQ
_C = 128
_PER_W = _N // _NW // _C
_NB = 3


def _make_kernel():
    mesh = plsc.VectorSubcoreMesh(core_axis_name="c", subcore_axis_name="s")

    @functools.partial(
        pl.kernel,
        mesh=mesh,
        out_type=jax.ShapeDtypeStruct((_N, D_MODEL), jnp.float32),
        scratch_types=(
            [pltpu.VMEM((_PER_W, _C), jnp.int32)]
            + [pltpu.VMEM((_C,), jnp.int32) for _ in range(_NB)]
            + [pltpu.VMEM((_C, 2 * D_MODEL), jnp.float32) for _ in range(_NB)]
            + [pltpu.VMEM((_C, D_MODEL), jnp.float32) for _ in range(_NB)]
            + [pltpu.SemaphoreType.DMA for _ in range(2 * _NB)]
        ),
        compiler_params=pltpu.CompilerParams(use_tc_tiling_on_sc=True,
                                             needs_layout_passes=False),
    )
    def body(idx_hbm, table_hbm, out_hbm, idx_v, *rest):
        ih = rest[:_NB]
        gb = rest[_NB:2 * _NB]
        st = rest[2 * _NB:3 * _NB]
        sg = rest[3 * _NB:3 * _NB + _NB]
        sw = rest[3 * _NB + _NB:]
        wid = lax.axis_index("s") * _NC + lax.axis_index("c")

        pltpu.sync_copy(idx_hbm.at[wid], idx_v)

        def prep(r, b):
            for v in range(_C // _L):
                ih[b][pl.ds(v * _L, _L)] = idx_v[r, pl.ds(v * _L, _L)]

        def g_start(b):
            pltpu.async_copy(table_hbm.at[ih[b]], gb[b], sg[b])

        def g_wait(b):
            pltpu.make_async_copy(table_hbm.at[ih[b]], gb[b], sg[b]).wait()

        def out_slc(r):
            base = pl.multiple_of(wid * (_PER_W * _C) + r * _C, 8)
            return out_hbm.at[pl.ds(base, _C), :]

        def w_start(r, b):
            pltpu.async_copy(st[b], out_slc(r), sw[b])

        def w_wait(r, b):
            pltpu.make_async_copy(st[b], out_slc(r), sw[b]).wait()

        def compact64(b):
            def jbody(j2, carry):
                j0 = j2 * 2
                vals = []
                for jj in range(2):
                    for v in range(D_MODEL // _L):
                        vals.append(gb[b][j0 + jj, pl.ds(v * _L, _L)])
                i = 0
                for jj in range(2):
                    for v in range(D_MODEL // _L):
                        st[b][j0 + jj, pl.ds(v * _L, _L)] = vals[i]
                        i += 1
                return carry

            lax.fori_loop(0, _C // 2, jbody, 0)

        for b in range(_NB):
            prep(b, b)
            g_start(b)

        def step(j, carry):
            base = j * _NB
            for b in range(_NB):
                r = base + b
                g_wait(b)
                compact64(b)
                w_start(r, b)
                pb = (b - 1) % _NB
                pr = r + _NB - 1

                @pl.when(jnp.logical_and(pr >= _NB, pr < _PER_W))
                def _():
                    w_wait(pr - _NB, pb)
                    prep(pr, pb)
                    g_start(pb)
            return carry

        lax.fori_loop(0, _PER_W // _NB, step, 0)

        for r in range(_PER_W - _PER_W % _NB, _PER_W):
            b = r % _NB
            g_wait(b)
            compact64(b)
            w_start(r, b)
        for r in range(_PER_W - _NB, _PER_W):
            w_wait(r, r % _NB)

    return body


_sc_kernel = _make_kernel()


def kernel(inp, table):
    idx3 = jnp.transpose(inp).reshape(_NW, _PER_W, _C)
    tableP = jnp.pad(table, ((0, 0), (0, D_MODEL)))
    out = _sc_kernel(idx3, tableP)
    return out.reshape(SEQ, BATCH, D_MODEL)


# speedup vs baseline: 1.8676x; 1.0007x over previous
import functools

import jax
import jax.numpy as jnp
from jax import lax
from jax.experimental import pallas as pl
from jax.experimental.pallas import tpu as pltpu
from jax.experimental.pallas import tpu_sc as plsc

VOCAB = 1000000
D_MODEL = 64
BATCH = 4096
SEQ = 200

_INFO = plsc.get_sparse_core_info()
_NC = _INFO.num_cores
_NS = _INFO.num_subcores
_NW = _NC * _NS
_L = 16

_N = BATCH * SEQ
_C = 128
_PER_W = _N // _NW // _C
_NB = 3


def _make_kernel():
    mesh = plsc.VectorSubcoreMesh(core_axis_name="c", subcore_axis_name="s")

    @functools.partial(
        pl.kernel,
        mesh=mesh,
        out_type=jax.ShapeDtypeStruct((_N, D_MODEL), jnp.float32),
        scratch_types=(
            [pltpu.VMEM((_PER_W, _C), jnp.int32)]
            + [pltpu.VMEM((_C,), jnp.int32) for _ in range(_NB)]
            + [pltpu.VMEM((_C, 2 * D_MODEL), jnp.float32) for _ in range(_NB)]
            + [pltpu.VMEM((_C, D_MODEL), jnp.float32) for _ in range(_NB)]
            + [pltpu.SemaphoreType.DMA for _ in range(2 * _NB)]
        ),
        compiler_params=pltpu.CompilerParams(use_tc_tiling_on_sc=True,
                                             needs_layout_passes=False),
    )
    def body(idx_hbm, table_hbm, out_hbm, idx_v, *rest):
        ih = rest[:_NB]
        gb = rest[_NB:2 * _NB]
        st = rest[2 * _NB:3 * _NB]
        sg = rest[3 * _NB:3 * _NB + _NB]
        sw = rest[3 * _NB + _NB:]
        wid = lax.axis_index("s") * _NC + lax.axis_index("c")

        pltpu.sync_copy(idx_hbm.at[wid], idx_v)

        def g_start(r, b):
            pltpu.async_copy(table_hbm.at[idx_v.at[r]], gb[b], sg[b])

        def g_wait(r, b):
            pltpu.make_async_copy(table_hbm.at[idx_v.at[r]], gb[b],
                                  sg[b]).wait()

        def out_slc(r):
            base = pl.multiple_of(wid * (_PER_W * _C) + r * _C, 8)
            return out_hbm.at[pl.ds(base, _C), :]

        def w_start(r, b):
            pltpu.async_copy(st[b], out_slc(r), sw[b])

        def w_wait(r, b):
            pltpu.make_async_copy(st[b], out_slc(r), sw[b]).wait()

        def compact64(b):
            def jbody(j2, carry):
                j0 = j2 * 2
                vals = []
                for jj in range(2):
                    for v in range(D_MODEL // _L):
                        vals.append(gb[b][j0 + jj, pl.ds(v * _L, _L)])
                i = 0
                for jj in range(2):
                    for v in range(D_MODEL // _L):
                        st[b][j0 + jj, pl.ds(v * _L, _L)] = vals[i]
                        i += 1
                return carry

            lax.fori_loop(0, _C // 2, jbody, 0)

        for b in range(_NB):
            g_start(b, b)

        def step(j, carry):
            base = j * _NB
            for b in range(_NB):
                r = base + b
                g_wait(r, b)
                compact64(b)
                w_start(r, b)
                pb = (b - 1) % _NB
                pr = r + _NB - 1

                @pl.when(jnp.logical_and(pr >= _NB, pr < _PER_W))
                def _():
                    w_wait(pr - _NB, pb)
                    g_start(pr, pb)
            return carry

        lax.fori_loop(0, _PER_W // _NB, step, 0)

        for r in range(_PER_W - _PER_W % _NB, _PER_W):
            b = r % _NB
            g_wait(r, b)
            compact64(b)
            w_start(r, b)
        for r in range(_PER_W - _NB, _PER_W):
            w_wait(r, r % _NB)

    return body


_sc_kernel = _make_kernel()


def kernel(inp, table):
    idx3 = jnp.transpose(inp).reshape(_NW, _PER_W, _C)
    tableP = jnp.pad(table, ((0, 0), (0, D_MODEL)))
    out = _sc_kernel(idx3, tableP)
    return out.reshape(SEQ, BATCH, D_MODEL)


# speedup vs baseline: 1.8730x; 1.0029x over previous
import functools

import jax
import jax.numpy as jnp
from jax import lax
from jax.experimental import pallas as pl
from jax.experimental.pallas import tpu as pltpu
from jax.experimental.pallas import tpu_sc as plsc

VOCAB = 1000000
D_MODEL = 64
BATCH = 4096
SEQ = 200

_INFO = plsc.get_sparse_core_info()
_NC = _INFO.num_cores
_NS = _INFO.num_subcores
_NW = _NC * _NS
_L = 16

_N = BATCH * SEQ
_C = 128
_PER_W = _N // _NW // _C
_NB = 3


def _make_kernel():
    mesh = plsc.VectorSubcoreMesh(core_axis_name="c", subcore_axis_name="s")

    @functools.partial(
        pl.kernel,
        mesh=mesh,
        out_type=jax.ShapeDtypeStruct((_N, D_MODEL), jnp.float32),
        scratch_types=(
            [pltpu.VMEM((_PER_W, _C), jnp.int32)]
            + [pltpu.VMEM((_C, 2 * D_MODEL), jnp.float32) for _ in range(_NB)]
            + [pltpu.VMEM((_C, D_MODEL), jnp.float32) for _ in range(_NB)]
            + [pltpu.SemaphoreType.DMA for _ in range(2 * _NB)]
        ),
        compiler_params=pltpu.CompilerParams(use_tc_tiling_on_sc=True,
                                             needs_layout_passes=False),
    )
    def body(idx_hbm, table_hbm, out_hbm, idx_v, *rest):
        gb = rest[:_NB]
        st = rest[_NB:2 * _NB]
        sg = rest[2 * _NB:3 * _NB]
        sw = rest[3 * _NB:]
        wid = lax.axis_index("s") * _NC + lax.axis_index("c")

        pltpu.sync_copy(idx_hbm.at[wid], idx_v)

        def g_start(r, b):
            pltpu.async_copy(table_hbm.at[idx_v.at[r]], gb[b], sg[b])

        def g_wait(r, b):
            pltpu.make_async_copy(table_hbm.at[idx_v.at[r]], gb[b],
                                  sg[b]).wait()

        def out_slc(r):
            base = pl.multiple_of(wid * (_PER_W * _C) + r * _C, 8)
            return out_hbm.at[pl.ds(base, _C), :]

        def w_start(r, b):
            pltpu.async_copy(st[b], out_slc(r), sw[b])

        def w_wait(r, b):
            pltpu.make_async_copy(st[b], out_slc(r), sw[b]).wait()

        def compact64(b):
            def jbody(j2, carry):
                j0 = j2 * 2
                vals = []
                for jj in range(2):
                    for v in range(D_MODEL // _L):
                        vals.append(gb[b][j0 + jj, pl.ds(v * _L, _L)])
                i = 0
                for jj in range(2):
                    for v in range(D_MODEL // _L):
                        st[b][j0 + jj, pl.ds(v * _L, _L)] = vals[i]
                        i += 1
                return carry

            lax.fori_loop(0, _C // 2, jbody, 0)

        for b in range(_NB):
            g_start(b, b)

        def step(j, carry):
            base = j * _NB
            for b in range(_NB):
                r = base + b
                g_wait(r, b)
                compact64(b)
                w_start(r, b)
                pb = (b - 1) % _NB
                pr = r + _NB - 1

                @pl.when(jnp.logical_and(pr >= _NB, pr < _PER_W))
                def _():
                    w_wait(pr - _NB, pb)
                    g_start(pr, pb)
            return carry

        lax.fori_loop(0, _PER_W // _NB, step, 0)

        for r in range(_PER_W - _PER_W % _NB, _PER_W):
            b = r % _NB
            g_wait(r, b)
            compact64(b)
            w_start(r, b)
        for r in range(_PER_W - _NB, _PER_W):
            w_wait(r, r % _NB)

    return body


_sc_kernel = _make_kernel()


def kernel(inp, table):
    idx3 = jnp.transpose(inp).reshape(_NW, _PER_W, _C)
    tableP = jnp.pad(table, ((0, 0), (0, D_MODEL)))
    out = _sc_kernel(idx3, tableP)
    return out.reshape(SEQ, BATCH, D_MODEL)
